# Initial kernel scaffold; baseline (speedup 1.0000x reference)
import jax
import jax.numpy as jnp
from jax.experimental import pallas as pl


def kernel(boxes, scores):
    raise NotImplementedError("write your pallas kernel here")


# speedup vs baseline: 25.3233x; 25.3233x over previous
import jax
import jax.numpy as jnp
from jax.experimental import pallas as pl
from jax.experimental.pallas import tpu as pltpu

_N = 5000
_B = 128
_NPAD = 5120
_NB = _NPAD // _B
_T = 0.3
_EPS = 1e-9


def _nms_body(bc_ref, br_ref, out_ref, keep_ref):
    cx = br_ref[0:1, :]
    cy = br_ref[1:2, :]
    w = br_ref[2:3, :]
    h = br_ref[3:4, :]
    s = br_ref[4:5, :]
    x1r = cx - w * 0.5
    y1r = cy - h * 0.5
    x2r = cx + w * 0.5
    y2r = cy + h * 0.5
    area_r = jnp.maximum(x2r - x1r, 0.0) * jnp.maximum(y2r - y1r, 0.0)

    col = jax.lax.broadcasted_iota(jnp.int32, (1, _NPAD), 1)
    keep_ref[...] = (col < _N).astype(jnp.float32)

    iota_b = jax.lax.broadcasted_iota(jnp.int32, (1, _B), 1)

    def block_step(bi, carry):
        start = pl.multiple_of(bi * _B, _B)

        blk = bc_ref[pl.ds(start, _B), :]
        cxc = jax.lax.slice(blk, (0, 0), (_B, 1))
        cyc = jax.lax.slice(blk, (0, 1), (_B, 2))
        wc = jax.lax.slice(blk, (0, 2), (_B, 3))
        hc = jax.lax.slice(blk, (0, 3), (_B, 4))
        x1c = cxc - wc * 0.5
        y1c = cyc - hc * 0.5
        x2c = cxc + wc * 0.5
        y2c = cyc + hc * 0.5
        area_c = jnp.maximum(x2c - x1c, 0.0) * jnp.maximum(y2c - y1c, 0.0)

        xx1 = jnp.maximum(x1c, x1r)
        yy1 = jnp.maximum(y1c, y1r)
        xx2 = jnp.minimum(x2c, x2r)
        yy2 = jnp.minimum(y2c, y2r)
        inter = jnp.maximum(xx2 - xx1, 0.0) * jnp.maximum(yy2 - yy1, 0.0)
        union = area_c + area_r - inter
        ovl = (inter > _T * (union + _EPS)).astype(jnp.float32)

        rcx = br_ref[0:1, pl.ds(start, _B)]
        rcy = br_ref[1:2, pl.ds(start, _B)]
        rw = br_ref[2:3, pl.ds(start, _B)]
        rh = br_ref[3:4, pl.ds(start, _B)]
        bx1 = rcx - rw * 0.5
        by1 = rcy - rh * 0.5
        bx2 = rcx + rw * 0.5
        by2 = rcy + rh * 0.5
        barea = jnp.maximum(bx2 - bx1, 0.0) * jnp.maximum(by2 - by1, 0.0)
        bxx1 = jnp.maximum(x1c, bx1)
        byy1 = jnp.maximum(y1c, by1)
        bxx2 = jnp.minimum(x2c, bx2)
        byy2 = jnp.minimum(y2c, by2)
        binter = jnp.maximum(bxx2 - bxx1, 0.0) * jnp.maximum(byy2 - byy1, 0.0)
        bunion = area_c + barea - binter
        ovl_bb = (binter > _T * (bunion + _EPS)).astype(jnp.float32)

        kb = keep_ref[0:1, pl.ds(start, _B)]
        for j in range(_B - 1):
            kj = jax.lax.slice(kb, (0, j), (1, j + 1))
            row = jax.lax.slice(ovl_bb, (j, 0), (j + 1, _B))
            m = (iota_b > j).astype(jnp.float32)
            kb = kb * (1.0 - row * m * kj)
        keep_ref[0:1, pl.ds(start, _B)] = kb

        cnt = jax.lax.dot_general(
            kb, ovl, (((1,), (0,)), ((), ())),
            preferred_element_type=jnp.float32)
        later = (col >= start + _B).astype(jnp.float32)
        keep_ref[...] = keep_ref[...] * (1.0 - jnp.minimum(cnt, 1.0) * later)
        return carry

    jax.lax.fori_loop(0, _NB, block_step, 0)
    keep = keep_ref[...]

    zero = jnp.zeros((1, _NPAD), jnp.float32)
    out_ref[...] = jnp.concatenate(
        [s * keep, x1r * keep, y1r * keep, x2r * keep, y2r * keep,
         zero, zero, zero], axis=0)


def kernel(boxes, scores):
    order = jnp.argsort(-scores)
    b = jnp.take(boxes, order, axis=0)
    s = jnp.take(scores, order, axis=0)
    bc = jnp.zeros((_NPAD, 8), jnp.float32)
    bc = bc.at[:_N, :4].set(b)
    bc = bc.at[:_N, 4].set(s)
    br = bc.T
    out = pl.pallas_call(
        _nms_body,
        out_shape=jax.ShapeDtypeStruct((8, _NPAD), jnp.float32),
        scratch_shapes=[
            pltpu.VMEM((1, _NPAD), jnp.float32),
        ],
    )(bc, br)
    return out[0:5, :_N].T


# speedup vs baseline: 75.5994x; 2.9854x over previous
import jax
import jax.numpy as jnp
from jax.experimental import pallas as pl
from jax.experimental.pallas import tpu as pltpu

_N = 5000
_B = 128
_NPAD = 5120
_NB = _NPAD // _B
_T = 0.3
_EPS = 1e-9


def _nms_body(bc_ref, br_ref, out_ref, keep_ref):
    cx = br_ref[0:1, :]
    cy = br_ref[1:2, :]
    w = br_ref[2:3, :]
    h = br_ref[3:4, :]
    s = br_ref[4:5, :]
    x1r = cx - w * 0.5
    y1r = cy - h * 0.5
    x2r = cx + w * 0.5
    y2r = cy + h * 0.5
    area_r = jnp.maximum(x2r - x1r, 0.0) * jnp.maximum(y2r - y1r, 0.0)

    col = jax.lax.broadcasted_iota(jnp.int32, (1, _NPAD), 1)
    keep_ref[...] = (col < _N).astype(jnp.float32)

    iota_b = jax.lax.broadcasted_iota(jnp.int32, (1, _B), 1)

    def block_step(bi, carry):
        start = pl.multiple_of(bi * _B, _B)

        blk = bc_ref[pl.ds(start, _B), :]
        cxc = jax.lax.slice(blk, (0, 0), (_B, 1))
        cyc = jax.lax.slice(blk, (0, 1), (_B, 2))
        wc = jax.lax.slice(blk, (0, 2), (_B, 3))
        hc = jax.lax.slice(blk, (0, 3), (_B, 4))
        x1c = cxc - wc * 0.5
        y1c = cyc - hc * 0.5
        x2c = cxc + wc * 0.5
        y2c = cyc + hc * 0.5
        area_c = jnp.maximum(x2c - x1c, 0.0) * jnp.maximum(y2c - y1c, 0.0)

        xx1 = jnp.maximum(x1c, x1r)
        yy1 = jnp.maximum(y1c, y1r)
        xx2 = jnp.minimum(x2c, x2r)
        yy2 = jnp.minimum(y2c, y2r)
        inter = jnp.maximum(xx2 - xx1, 0.0) * jnp.maximum(yy2 - yy1, 0.0)
        union = area_c + area_r - inter
        ovl = (inter > _T * (union + _EPS)).astype(jnp.float32)

        rcx = br_ref[0:1, pl.ds(start, _B)]
        rcy = br_ref[1:2, pl.ds(start, _B)]
        rw = br_ref[2:3, pl.ds(start, _B)]
        rh = br_ref[3:4, pl.ds(start, _B)]
        bx1 = rcx - rw * 0.5
        by1 = rcy - rh * 0.5
        bx2 = rcx + rw * 0.5
        by2 = rcy + rh * 0.5
        barea = jnp.maximum(bx2 - bx1, 0.0) * jnp.maximum(by2 - by1, 0.0)
        bxx1 = jnp.maximum(x1c, bx1)
        byy1 = jnp.maximum(y1c, by1)
        bxx2 = jnp.minimum(x2c, bx2)
        byy2 = jnp.minimum(y2c, by2)
        binter = jnp.maximum(bxx2 - bxx1, 0.0) * jnp.maximum(byy2 - byy1, 0.0)
        bunion = area_c + barea - binter
        ovl_bb = (binter > _T * (bunion + _EPS)).astype(jnp.float32)

        row_i = jax.lax.broadcasted_iota(jnp.int32, (_B, _B), 0)
        col_i = jax.lax.broadcasted_iota(jnp.int32, (_B, _B), 1)
        m_tri = ovl_bb * (row_i < col_i).astype(jnp.float32)

        kb0 = keep_ref[0:1, pl.ds(start, _B)]

        def nxt(k):
            cnt = jax.lax.dot_general(
                k, m_tri, (((1,), (0,)), ((), ())),
                preferred_element_type=jnp.float32)
            return kb0 * (cnt < 0.5).astype(jnp.float32)

        def cond(c):
            k, kn = c
            return jnp.sum(jnp.abs(k - kn)) > 0.0

        def body(c):
            _, k = c
            return (k, nxt(k))

        k0 = kb0
        _, kb = jax.lax.while_loop(cond, body, (k0, nxt(k0)))
        keep_ref[0:1, pl.ds(start, _B)] = kb

        cnt = jax.lax.dot_general(
            kb, ovl, (((1,), (0,)), ((), ())),
            preferred_element_type=jnp.float32)
        later = (col >= start + _B).astype(jnp.float32)
        keep_ref[...] = keep_ref[...] * (1.0 - jnp.minimum(cnt, 1.0) * later)
        return carry

    jax.lax.fori_loop(0, _NB, block_step, 0)
    keep = keep_ref[...]

    zero = jnp.zeros((1, _NPAD), jnp.float32)
    out_ref[...] = jnp.concatenate(
        [s * keep, x1r * keep, y1r * keep, x2r * keep, y2r * keep,
         zero, zero, zero], axis=0)


def kernel(boxes, scores):
    order = jnp.argsort(-scores)
    b = jnp.take(boxes, order, axis=0)
    s = jnp.take(scores, order, axis=0)
    bc = jnp.zeros((_NPAD, 8), jnp.float32)
    bc = bc.at[:_N, :4].set(b)
    bc = bc.at[:_N, 4].set(s)
    br = bc.T
    out = pl.pallas_call(
        _nms_body,
        out_shape=jax.ShapeDtypeStruct((8, _NPAD), jnp.float32),
        scratch_shapes=[
            pltpu.VMEM((1, _NPAD), jnp.float32),
        ],
    )(bc, br)
    return out[0:5, :_N].T


# speedup vs baseline: 91.4924x; 1.2102x over previous
import jax
import jax.numpy as jnp
from jax.experimental import pallas as pl
from jax.experimental.pallas import tpu as pltpu

_N = 5000
_B = 128
_NPAD = 5120
_NB = _NPAD // _B
_NSEG = 4
_BPS = _NB // _NSEG
_T = 0.3
_EPS = 1e-9


def _nms_body(bc_ref, br_ref, out_ref, keep_ref):
    cx = br_ref[0:1, :]
    cy = br_ref[1:2, :]
    w = br_ref[2:3, :]
    h = br_ref[3:4, :]
    s = br_ref[4:5, :]
    x1r = cx - w * 0.5
    y1r = cy - h * 0.5
    x2r = cx + w * 0.5
    y2r = cy + h * 0.5
    area_r = jnp.maximum(x2r - x1r, 0.0) * jnp.maximum(y2r - y1r, 0.0)

    col = jax.lax.broadcasted_iota(jnp.int32, (1, _NPAD), 1)
    keep_ref[...] = (col < _N).astype(jnp.float32)

    row_i = jax.lax.broadcasted_iota(jnp.int32, (_B, _B), 0)
    col_i = jax.lax.broadcasted_iota(jnp.int32, (_B, _B), 1)
    tri = (row_i < col_i).astype(jnp.float32)

    def make_block_step(c_off):
        wd = _NPAD - c_off
        x1s = jax.lax.slice(x1r, (0, c_off), (1, _NPAD))
        y1s = jax.lax.slice(y1r, (0, c_off), (1, _NPAD))
        x2s = jax.lax.slice(x2r, (0, c_off), (1, _NPAD))
        y2s = jax.lax.slice(y2r, (0, c_off), (1, _NPAD))
        areas = jax.lax.slice(area_r, (0, c_off), (1, _NPAD))
        cols = jax.lax.slice(col, (0, c_off), (1, _NPAD))

        def block_step(bi, carry):
            start = pl.multiple_of(bi * _B, _B)

            blk = bc_ref[pl.ds(start, _B), :]
            cxc = jax.lax.slice(blk, (0, 0), (_B, 1))
            cyc = jax.lax.slice(blk, (0, 1), (_B, 2))
            wc = jax.lax.slice(blk, (0, 2), (_B, 3))
            hc = jax.lax.slice(blk, (0, 3), (_B, 4))
            x1c = cxc - wc * 0.5
            y1c = cyc - hc * 0.5
            x2c = cxc + wc * 0.5
            y2c = cyc + hc * 0.5
            area_c = jnp.maximum(x2c - x1c, 0.0) * jnp.maximum(y2c - y1c, 0.0)

            xx1 = jnp.maximum(x1c, x1s)
            yy1 = jnp.maximum(y1c, y1s)
            xx2 = jnp.minimum(x2c, x2s)
            yy2 = jnp.minimum(y2c, y2s)
            inter = jnp.maximum(xx2 - xx1, 0.0) * jnp.maximum(yy2 - yy1, 0.0)
            union = area_c + areas - inter
            ovl = (inter / (union + _EPS) > _T).astype(jnp.float32)

            rcx = br_ref[0:1, pl.ds(start, _B)]
            rcy = br_ref[1:2, pl.ds(start, _B)]
            rw = br_ref[2:3, pl.ds(start, _B)]
            rh = br_ref[3:4, pl.ds(start, _B)]
            bx1 = rcx - rw * 0.5
            by1 = rcy - rh * 0.5
            bx2 = rcx + rw * 0.5
            by2 = rcy + rh * 0.5
            barea = jnp.maximum(bx2 - bx1, 0.0) * jnp.maximum(by2 - by1, 0.0)
            bxx1 = jnp.maximum(x1c, bx1)
            byy1 = jnp.maximum(y1c, by1)
            bxx2 = jnp.minimum(x2c, bx2)
            byy2 = jnp.minimum(y2c, by2)
            binter = (jnp.maximum(bxx2 - bxx1, 0.0)
                      * jnp.maximum(byy2 - byy1, 0.0))
            bunion = area_c + barea - binter
            m_tri = (binter / (bunion + _EPS) > _T).astype(jnp.float32) * tri

            kb0 = keep_ref[0:1, pl.ds(start, _B)]

            def nxt(k):
                cnt = jax.lax.dot_general(
                    k, m_tri, (((1,), (0,)), ((), ())),
                    preferred_element_type=jnp.float32)
                return kb0 * (cnt < 0.5).astype(jnp.float32)

            def cond(c):
                k, kn = c
                return jnp.sum(jnp.abs(k - kn)) > 0.0

            def body(c):
                _, k = c
                return (k, nxt(k))

            _, kb = jax.lax.while_loop(cond, body, (kb0, nxt(kb0)))
            keep_ref[0:1, pl.ds(start, _B)] = kb

            cnt = jax.lax.dot_general(
                kb, ovl, (((1,), (0,)), ((), ())),
                preferred_element_type=jnp.float32)
            later = (cols >= start + _B).astype(jnp.float32)
            kw = keep_ref[0:1, c_off:_NPAD]
            keep_ref[0:1, c_off:_NPAD] = (
                kw * (1.0 - jnp.minimum(cnt, 1.0) * later))
            return carry

        return block_step

    for seg in range(_NSEG):
        jax.lax.fori_loop(seg * _BPS, (seg + 1) * _BPS,
                          make_block_step(seg * _BPS * _B), 0)

    keep = keep_ref[...]
    zero = jnp.zeros((1, _NPAD), jnp.float32)
    out_ref[...] = jnp.concatenate(
        [s * keep, x1r * keep, y1r * keep, x2r * keep, y2r * keep,
         zero, zero, zero], axis=0)


def kernel(boxes, scores):
    order = jnp.argsort(-scores)
    b = jnp.take(boxes, order, axis=0)
    s = jnp.take(scores, order, axis=0)
    bc = jnp.zeros((_NPAD, 8), jnp.float32)
    bc = bc.at[:_N, :4].set(b)
    bc = bc.at[:_N, 4].set(s)
    br = bc.T
    out = pl.pallas_call(
        _nms_body,
        out_shape=jax.ShapeDtypeStruct((8, _NPAD), jnp.float32),
        scratch_shapes=[
            pltpu.VMEM((1, _NPAD), jnp.float32),
        ],
    )(bc, br)
    return out[0:5, :_N].T


# speedup vs baseline: 102.5933x; 1.1213x over previous
import functools

import jax
import jax.numpy as jnp
from jax import lax
from jax.experimental import pallas as pl
from jax.experimental.pallas import tpu as pltpu
from jax.experimental.pallas import tpu_sc as plsc

_N = 5000
_B = 128
_NPAD = 5120
_NB = _NPAD // _B
_NSEG = 4
_BPS = _NB // _NSEG
_T = 0.3
_EPS = 1e-9

_NW = 32
_L = 16


def _reorder_body(boxesf_hbm, scores_hbm, order_hbm, brf_hbm,
                  order_v, idxc_v, row_v, sem):
    wid = lax.axis_index("s") * 2 + lax.axis_index("c")

    def do_block(base):
        pltpu.sync_copy(order_hbm.at[pl.ds(base, _B)], order_v)
        for c in range(4):
            for g in range(_B // _L):
                idx = order_v[pl.ds(g * _L, _L)]
                idxc_v[pl.ds(g * _L, _L)] = idx * 4 + c
            pltpu.async_copy(boxesf_hbm.at[idxc_v], row_v, sem).wait()
            pltpu.sync_copy(row_v, brf_hbm.at[pl.ds(c * _NPAD + base, _B)])
        pltpu.async_copy(scores_hbm.at[order_v], row_v, sem).wait()
        pltpu.sync_copy(row_v, brf_hbm.at[pl.ds(4 * _NPAD + base, _B)])

    do_block(wid * _B)

    @pl.when(wid < _NB - _NW)
    def _():
        do_block((wid + _NW) * _B)


_reorder = functools.partial(
    pl.kernel,
    out_type=jax.ShapeDtypeStruct((5 * _NPAD,), jnp.float32),
    mesh=plsc.VectorSubcoreMesh(core_axis_name="c", subcore_axis_name="s"),
    scratch_types=[
        pltpu.VMEM((_B,), jnp.int32),
        pltpu.VMEM((_B,), jnp.int32),
        pltpu.VMEM((_B,), jnp.float32),
        pltpu.SemaphoreType.DMA,
    ],
)(_reorder_body)


def _nms_body(br_ref, out_ref, keep_ref):
    cx = br_ref[0:1, :]
    cy = br_ref[1:2, :]
    w = br_ref[2:3, :]
    h = br_ref[3:4, :]
    s = br_ref[4:5, :]
    x1r = cx - w * 0.5
    y1r = cy - h * 0.5
    x2r = cx + w * 0.5
    y2r = cy + h * 0.5
    area_r = jnp.maximum(x2r - x1r, 0.0) * jnp.maximum(y2r - y1r, 0.0)

    col = jax.lax.broadcasted_iota(jnp.int32, (1, _NPAD), 1)
    keep_ref[...] = (col < _N).astype(jnp.float32)

    row_i = jax.lax.broadcasted_iota(jnp.int32, (_B, _B), 0)
    col_i = jax.lax.broadcasted_iota(jnp.int32, (_B, _B), 1)
    tri = (row_i < col_i).astype(jnp.float32)

    def make_block_step(c_off):
        wd = _NPAD - c_off
        x1s = jax.lax.slice(x1r, (0, c_off), (1, _NPAD))
        y1s = jax.lax.slice(y1r, (0, c_off), (1, _NPAD))
        x2s = jax.lax.slice(x2r, (0, c_off), (1, _NPAD))
        y2s = jax.lax.slice(y2r, (0, c_off), (1, _NPAD))
        areas = jax.lax.slice(area_r, (0, c_off), (1, _NPAD))
        cols = jax.lax.slice(col, (0, c_off), (1, _NPAD))

        def block_step(bi, carry):
            start = pl.multiple_of(bi * _B, _B)

            rows4 = br_ref[0:4, pl.ds(start, _B)]
            blk = jnp.transpose(rows4, (1, 0))
            cxc = jax.lax.slice(blk, (0, 0), (_B, 1))
            cyc = jax.lax.slice(blk, (0, 1), (_B, 2))
            wc = jax.lax.slice(blk, (0, 2), (_B, 3))
            hc = jax.lax.slice(blk, (0, 3), (_B, 4))
            x1c = cxc - wc * 0.5
            y1c = cyc - hc * 0.5
            x2c = cxc + wc * 0.5
            y2c = cyc + hc * 0.5
            area_c = jnp.maximum(x2c - x1c, 0.0) * jnp.maximum(y2c - y1c, 0.0)

            xx1 = jnp.maximum(x1c, x1s)
            yy1 = jnp.maximum(y1c, y1s)
            xx2 = jnp.minimum(x2c, x2s)
            yy2 = jnp.minimum(y2c, y2s)
            inter = jnp.maximum(xx2 - xx1, 0.0) * jnp.maximum(yy2 - yy1, 0.0)
            union = area_c + areas - inter
            ovl = (inter / (union + _EPS) > _T).astype(jnp.float32)

            rcx = br_ref[0:1, pl.ds(start, _B)]
            rcy = br_ref[1:2, pl.ds(start, _B)]
            rw = br_ref[2:3, pl.ds(start, _B)]
            rh = br_ref[3:4, pl.ds(start, _B)]
            bx1 = rcx - rw * 0.5
            by1 = rcy - rh * 0.5
            bx2 = rcx + rw * 0.5
            by2 = rcy + rh * 0.5
            barea = jnp.maximum(bx2 - bx1, 0.0) * jnp.maximum(by2 - by1, 0.0)
            bxx1 = jnp.maximum(x1c, bx1)
            byy1 = jnp.maximum(y1c, by1)
            bxx2 = jnp.minimum(x2c, bx2)
            byy2 = jnp.minimum(y2c, by2)
            binter = (jnp.maximum(bxx2 - bxx1, 0.0)
                      * jnp.maximum(byy2 - byy1, 0.0))
            bunion = area_c + barea - binter
            m_tri = (binter / (bunion + _EPS) > _T).astype(jnp.float32) * tri

            kb0 = keep_ref[0:1, pl.ds(start, _B)]

            def nxt(k):
                cnt = jax.lax.dot_general(
                    k, m_tri, (((1,), (0,)), ((), ())),
                    preferred_element_type=jnp.float32)
                return kb0 * (cnt < 0.5).astype(jnp.float32)

            def cond(c):
                k, kn = c
                return jnp.sum(jnp.abs(k - kn)) > 0.0

            def body(c):
                _, k = c
                return (k, nxt(k))

            _, kb = jax.lax.while_loop(cond, body, (kb0, nxt(kb0)))
            keep_ref[0:1, pl.ds(start, _B)] = kb

            cnt = jax.lax.dot_general(
                kb, ovl, (((1,), (0,)), ((), ())),
                preferred_element_type=jnp.float32)
            later = (cols >= start + _B).astype(jnp.float32)
            kw = keep_ref[0:1, c_off:_NPAD]
            keep_ref[0:1, c_off:_NPAD] = (
                kw * (1.0 - jnp.minimum(cnt, 1.0) * later))
            return carry

        return block_step

    for seg in range(_NSEG):
        jax.lax.fori_loop(seg * _BPS, (seg + 1) * _BPS,
                          make_block_step(seg * _BPS * _B), 0)

    keep = keep_ref[...]
    zero = jnp.zeros((1, _NPAD), jnp.float32)
    out_ref[...] = jnp.concatenate(
        [s * keep, x1r * keep, y1r * keep, x2r * keep, y2r * keep,
         zero, zero, zero], axis=0)


def kernel(boxes, scores):
    order = jnp.argsort(-scores).astype(jnp.int32)
    order_p = jnp.zeros((_NPAD,), jnp.int32).at[:_N].set(order)
    brf = _reorder(boxes.reshape(-1), scores, order_p)
    br = brf.reshape(5, _NPAD)
    out = pl.pallas_call(
        _nms_body,
        out_shape=jax.ShapeDtypeStruct((8, _NPAD), jnp.float32),
        scratch_shapes=[
            pltpu.VMEM((1, _NPAD), jnp.float32),
        ],
    )(br)
    return out[0:5, :_N].T


# speedup vs baseline: 112.3334x; 1.0949x over previous
import functools

import jax
import jax.numpy as jnp
from jax import lax
from jax.experimental import pallas as pl
from jax.experimental.pallas import tpu as pltpu
from jax.experimental.pallas import tpu_sc as plsc

_N = 5000
_B = 128
_NPAD = 5120
_NB = _NPAD // _B
_NSEG = 8
_BPS = _NB // _NSEG
_T = 0.3
_EPS = 1e-9

_NW = 32
_L = 16


def _reorder_body(boxesf_hbm, scores_hbm, order_hbm, brf_hbm,
                  order_v, i0, i1, i2, i3, r0, r1, r2, r3, r4, sem, sem2):
    idxc_v = [i0, i1, i2, i3]
    row_v = [r0, r1, r2, r3, r4]
    wid = lax.axis_index("s") * 2 + lax.axis_index("c")

    def do_block(base):
        pltpu.sync_copy(order_hbm.at[pl.ds(base, _B)], order_v)
        for c in range(4):
            for g in range(_B // _L):
                idx = order_v[pl.ds(g * _L, _L)]
                idxc_v[c][pl.ds(g * _L, _L)] = idx * 4 + c
        gathers = [
            pltpu.async_copy(boxesf_hbm.at[idxc_v[c]], row_v[c], sem)
            for c in range(4)
        ] + [pltpu.async_copy(scores_hbm.at[order_v], row_v[4], sem)]
        for cp in gathers:
            cp.wait()
        outs = [
            pltpu.async_copy(row_v[c],
                             brf_hbm.at[pl.ds(c * _NPAD + base, _B)], sem2)
            for c in range(5)
        ]
        for cp in outs:
            cp.wait()

    do_block(wid * _B)

    @pl.when(wid < _NB - _NW)
    def _():
        do_block((wid + _NW) * _B)


_reorder = functools.partial(
    pl.kernel,
    out_type=jax.ShapeDtypeStruct((5 * _NPAD,), jnp.float32),
    mesh=plsc.VectorSubcoreMesh(core_axis_name="c", subcore_axis_name="s"),
    scratch_types=(
        [pltpu.VMEM((_B,), jnp.int32)] * 5
        + [pltpu.VMEM((_B,), jnp.float32)] * 5
        + [pltpu.SemaphoreType.DMA, pltpu.SemaphoreType.DMA]
    ),
)(_reorder_body)


def _nms_body(br_ref, out_ref, keep_ref):
    cx = br_ref[0:1, :]
    cy = br_ref[1:2, :]
    w = br_ref[2:3, :]
    h = br_ref[3:4, :]
    s = br_ref[4:5, :]
    x1r = cx - w * 0.5
    y1r = cy - h * 0.5
    x2r = cx + w * 0.5
    y2r = cy + h * 0.5
    area_r = jnp.maximum(x2r - x1r, 0.0) * jnp.maximum(y2r - y1r, 0.0)

    col = jax.lax.broadcasted_iota(jnp.int32, (1, _NPAD), 1)
    keep_ref[...] = (col < _N).astype(jnp.float32)

    row_i = jax.lax.broadcasted_iota(jnp.int32, (_B, _B), 0)
    col_i = jax.lax.broadcasted_iota(jnp.int32, (_B, _B), 1)
    tri = (row_i < col_i).astype(jnp.float32)

    def make_block_step(c_off):
        wd = _NPAD - c_off
        x1s = jax.lax.slice(x1r, (0, c_off), (1, _NPAD))
        y1s = jax.lax.slice(y1r, (0, c_off), (1, _NPAD))
        x2s = jax.lax.slice(x2r, (0, c_off), (1, _NPAD))
        y2s = jax.lax.slice(y2r, (0, c_off), (1, _NPAD))
        areas = jax.lax.slice(area_r, (0, c_off), (1, _NPAD))
        cols = jax.lax.slice(col, (0, c_off), (1, _NPAD))

        def block_step(bi, carry):
            start = pl.multiple_of(bi * _B, _B)

            rows4 = br_ref[0:4, pl.ds(start, _B)]
            blk = jnp.transpose(rows4, (1, 0))
            cxc = jax.lax.slice(blk, (0, 0), (_B, 1))
            cyc = jax.lax.slice(blk, (0, 1), (_B, 2))
            wc = jax.lax.slice(blk, (0, 2), (_B, 3))
            hc = jax.lax.slice(blk, (0, 3), (_B, 4))
            x1c = cxc - wc * 0.5
            y1c = cyc - hc * 0.5
            x2c = cxc + wc * 0.5
            y2c = cyc + hc * 0.5
            area_c = jnp.maximum(x2c - x1c, 0.0) * jnp.maximum(y2c - y1c, 0.0)

            xx1 = jnp.maximum(x1c, x1s)
            yy1 = jnp.maximum(y1c, y1s)
            xx2 = jnp.minimum(x2c, x2s)
            yy2 = jnp.minimum(y2c, y2s)
            inter = jnp.maximum(xx2 - xx1, 0.0) * jnp.maximum(yy2 - yy1, 0.0)
            union = area_c + areas - inter
            ovl = (inter / (union + _EPS) > _T).astype(jnp.float32)

            rcx = br_ref[0:1, pl.ds(start, _B)]
            rcy = br_ref[1:2, pl.ds(start, _B)]
            rw = br_ref[2:3, pl.ds(start, _B)]
            rh = br_ref[3:4, pl.ds(start, _B)]
            bx1 = rcx - rw * 0.5
            by1 = rcy - rh * 0.5
            bx2 = rcx + rw * 0.5
            by2 = rcy + rh * 0.5
            barea = jnp.maximum(bx2 - bx1, 0.0) * jnp.maximum(by2 - by1, 0.0)
            bxx1 = jnp.maximum(x1c, bx1)
            byy1 = jnp.maximum(y1c, by1)
            bxx2 = jnp.minimum(x2c, bx2)
            byy2 = jnp.minimum(y2c, by2)
            binter = (jnp.maximum(bxx2 - bxx1, 0.0)
                      * jnp.maximum(byy2 - byy1, 0.0))
            bunion = area_c + barea - binter
            m_tri = (binter / (bunion + _EPS) > _T).astype(jnp.float32) * tri

            kb0 = keep_ref[0:1, pl.ds(start, _B)]

            def nxt(k):
                cnt = jax.lax.dot_general(
                    k, m_tri, (((1,), (0,)), ((), ())),
                    preferred_element_type=jnp.float32)
                return kb0 * (cnt < 0.5).astype(jnp.float32)

            def cond(c):
                k, kn = c
                return jnp.sum(jnp.abs(k - kn)) > 0.0

            def body(c):
                _, k = c
                return (k, nxt(k))

            _, kb = jax.lax.while_loop(cond, body, (kb0, nxt(kb0)))
            keep_ref[0:1, pl.ds(start, _B)] = kb

            cnt = jax.lax.dot_general(
                kb, ovl, (((1,), (0,)), ((), ())),
                preferred_element_type=jnp.float32)
            later = (cols >= start + _B).astype(jnp.float32)
            kw = keep_ref[0:1, c_off:_NPAD]
            keep_ref[0:1, c_off:_NPAD] = (
                kw * (1.0 - jnp.minimum(cnt, 1.0) * later))
            return carry

        return block_step

    for seg in range(_NSEG):
        jax.lax.fori_loop(seg * _BPS, (seg + 1) * _BPS,
                          make_block_step(seg * _BPS * _B), 0)

    keep = keep_ref[...]
    zero = jnp.zeros((1, _NPAD), jnp.float32)
    out_ref[...] = jnp.concatenate(
        [s * keep, x1r * keep, y1r * keep, x2r * keep, y2r * keep,
         zero, zero, zero], axis=0)


def kernel(boxes, scores):
    order = jnp.argsort(-scores).astype(jnp.int32)
    order_p = jnp.zeros((_NPAD,), jnp.int32).at[:_N].set(order)
    brf = _reorder(boxes.reshape(-1), scores, order_p)
    br = brf.reshape(5, _NPAD)
    out = pl.pallas_call(
        _nms_body,
        out_shape=jax.ShapeDtypeStruct((8, _NPAD), jnp.float32),
        scratch_shapes=[
            pltpu.VMEM((1, _NPAD), jnp.float32),
        ],
    )(br)
    return out[0:5, :_N].T


# speedup vs baseline: 123.7873x; 1.1020x over previous
import functools

import jax
import jax.numpy as jnp
from jax import lax
from jax.experimental import pallas as pl
from jax.experimental.pallas import tpu as pltpu
from jax.experimental.pallas import tpu_sc as plsc

_N = 5000
_B = 256
_NPAD = 5120
_NB = _NPAD // _B
_NSEG = 10
_BPS = _NB // _NSEG
_T = 0.3
_EPS = 1e-9

_NW = 32
_SCW = _NPAD // _NW
_L = 16


def _reorder_body(boxesf_hbm, scores_hbm, order_hbm, brf_hbm,
                  order_v, i0, i1, i2, i3, r0, r1, r2, r3, r4, sem, sem2):
    idxc_v = [i0, i1, i2, i3]
    row_v = [r0, r1, r2, r3, r4]
    wid = lax.axis_index("s") * 2 + lax.axis_index("c")
    base = wid * _SCW
    pltpu.sync_copy(order_hbm.at[pl.ds(base, _SCW)], order_v)
    for c in range(4):
        for g in range(_SCW // _L):
            idx = order_v[pl.ds(g * _L, _L)]
            idxc_v[c][pl.ds(g * _L, _L)] = idx * 4 + c
    gathers = [
        pltpu.async_copy(boxesf_hbm.at[idxc_v[c]], row_v[c], sem)
        for c in range(4)
    ] + [pltpu.async_copy(scores_hbm.at[order_v], row_v[4], sem)]
    for cp in gathers:
        cp.wait()
    outs = [
        pltpu.async_copy(row_v[c],
                         brf_hbm.at[pl.ds(c * _NPAD + base, _SCW)], sem2)
        for c in range(5)
    ]
    for cp in outs:
        cp.wait()


_reorder = functools.partial(
    pl.kernel,
    out_type=jax.ShapeDtypeStruct((5 * _NPAD,), jnp.float32),
    mesh=plsc.VectorSubcoreMesh(core_axis_name="c", subcore_axis_name="s"),
    scratch_types=(
        [pltpu.VMEM((_SCW,), jnp.int32)] * 5
        + [pltpu.VMEM((_SCW,), jnp.float32)] * 5
        + [pltpu.SemaphoreType.DMA, pltpu.SemaphoreType.DMA]
    ),
)(_reorder_body)


def _nms_body(br_ref, out_ref, keep_ref):
    cx = br_ref[0:1, :]
    cy = br_ref[1:2, :]
    w = br_ref[2:3, :]
    h = br_ref[3:4, :]
    s = br_ref[4:5, :]
    x1r = cx - w * 0.5
    y1r = cy - h * 0.5
    x2r = cx + w * 0.5
    y2r = cy + h * 0.5
    area_r = jnp.maximum(x2r - x1r, 0.0) * jnp.maximum(y2r - y1r, 0.0)

    col = jax.lax.broadcasted_iota(jnp.int32, (1, _NPAD), 1)
    keep_ref[...] = (col < _N).astype(jnp.float32)

    row_i = jax.lax.broadcasted_iota(jnp.int32, (_B, _B), 0)
    col_i = jax.lax.broadcasted_iota(jnp.int32, (_B, _B), 1)
    tri = (row_i < col_i).astype(jnp.float32)

    def make_block_step(c_off):
        wd = _NPAD - c_off
        x1s = jax.lax.slice(x1r, (0, c_off), (1, _NPAD))
        y1s = jax.lax.slice(y1r, (0, c_off), (1, _NPAD))
        x2s = jax.lax.slice(x2r, (0, c_off), (1, _NPAD))
        y2s = jax.lax.slice(y2r, (0, c_off), (1, _NPAD))
        areas = jax.lax.slice(area_r, (0, c_off), (1, _NPAD))
        cols = jax.lax.slice(col, (0, c_off), (1, _NPAD))

        def block_step(bi, carry):
            start = pl.multiple_of(bi * _B, _B)

            rows4 = br_ref[0:4, pl.ds(start, _B)]
            blk = jnp.transpose(rows4, (1, 0))
            cxc = jax.lax.slice(blk, (0, 0), (_B, 1))
            cyc = jax.lax.slice(blk, (0, 1), (_B, 2))
            wc = jax.lax.slice(blk, (0, 2), (_B, 3))
            hc = jax.lax.slice(blk, (0, 3), (_B, 4))
            x1c = cxc - wc * 0.5
            y1c = cyc - hc * 0.5
            x2c = cxc + wc * 0.5
            y2c = cyc + hc * 0.5
            area_c = jnp.maximum(x2c - x1c, 0.0) * jnp.maximum(y2c - y1c, 0.0)

            xx1 = jnp.maximum(x1c, x1s)
            yy1 = jnp.maximum(y1c, y1s)
            xx2 = jnp.minimum(x2c, x2s)
            yy2 = jnp.minimum(y2c, y2s)
            inter = jnp.maximum(xx2 - xx1, 0.0) * jnp.maximum(yy2 - yy1, 0.0)
            union = area_c + areas - inter
            ovl = (inter / (union + _EPS) > _T).astype(jnp.float32)

            rcx = br_ref[0:1, pl.ds(start, _B)]
            rcy = br_ref[1:2, pl.ds(start, _B)]
            rw = br_ref[2:3, pl.ds(start, _B)]
            rh = br_ref[3:4, pl.ds(start, _B)]
            bx1 = rcx - rw * 0.5
            by1 = rcy - rh * 0.5
            bx2 = rcx + rw * 0.5
            by2 = rcy + rh * 0.5
            barea = jnp.maximum(bx2 - bx1, 0.0) * jnp.maximum(by2 - by1, 0.0)
            bxx1 = jnp.maximum(x1c, bx1)
            byy1 = jnp.maximum(y1c, by1)
            bxx2 = jnp.minimum(x2c, bx2)
            byy2 = jnp.minimum(y2c, by2)
            binter = (jnp.maximum(bxx2 - bxx1, 0.0)
                      * jnp.maximum(byy2 - byy1, 0.0))
            bunion = area_c + barea - binter
            m_tri = (binter / (bunion + _EPS) > _T).astype(jnp.float32) * tri

            kb0 = keep_ref[0:1, pl.ds(start, _B)]

            def nxt(k):
                cnt = jax.lax.dot_general(
                    k, m_tri, (((1,), (0,)), ((), ())),
                    preferred_element_type=jnp.float32)
                return kb0 * (cnt < 0.5).astype(jnp.float32)

            def cond(c):
                k, kn = c
                return jnp.sum(jnp.abs(k - kn)) > 0.0

            def body(c):
                _, k = c
                return (k, nxt(k))

            _, kb = jax.lax.while_loop(cond, body, (kb0, nxt(kb0)))
            keep_ref[0:1, pl.ds(start, _B)] = kb

            cnt = jax.lax.dot_general(
                kb, ovl, (((1,), (0,)), ((), ())),
                preferred_element_type=jnp.float32)
            later = (cols >= start + _B).astype(jnp.float32)
            kw = keep_ref[0:1, c_off:_NPAD]
            keep_ref[0:1, c_off:_NPAD] = (
                kw * (1.0 - jnp.minimum(cnt, 1.0) * later))
            return carry

        return block_step

    for seg in range(_NSEG):
        jax.lax.fori_loop(seg * _BPS, (seg + 1) * _BPS,
                          make_block_step(seg * _BPS * _B), 0)

    keep = keep_ref[...]
    zero = jnp.zeros((1, _NPAD), jnp.float32)
    out_ref[...] = jnp.concatenate(
        [s * keep, x1r * keep, y1r * keep, x2r * keep, y2r * keep,
         zero, zero, zero], axis=0)


def kernel(boxes, scores):
    order = jnp.argsort(-scores).astype(jnp.int32)
    order_p = jnp.zeros((_NPAD,), jnp.int32).at[:_N].set(order)
    brf = _reorder(boxes.reshape(-1), scores, order_p)
    br = brf.reshape(5, _NPAD)
    out = pl.pallas_call(
        _nms_body,
        out_shape=jax.ShapeDtypeStruct((8, _NPAD), jnp.float32),
        scratch_shapes=[
            pltpu.VMEM((1, _NPAD), jnp.float32),
        ],
    )(br)
    return out[0:5, :_N].T


# speedup vs baseline: 129.8884x; 1.0493x over previous
import functools

import jax
import jax.numpy as jnp
from jax import lax
from jax.experimental import pallas as pl
from jax.experimental.pallas import tpu as pltpu
from jax.experimental.pallas import tpu_sc as plsc

_N = 5000
_B = 256
_NPAD = 5120
_NB = _NPAD // _B
_T = 0.3
_EPS = 1e-9

_NW = 32
_SCW = _NPAD // _NW
_L = 16


def _reorder_body(boxesf_hbm, scores_hbm, order_hbm, brf_hbm,
                  order_v, i0, i1, i2, i3, r0, r1, r2, r3, r4, sem, sem2):
    idxc_v = [i0, i1, i2, i3]
    row_v = [r0, r1, r2, r3, r4]
    wid = lax.axis_index("s") * 2 + lax.axis_index("c")
    base = wid * _SCW
    pltpu.sync_copy(order_hbm.at[pl.ds(base, _SCW)], order_v)
    for c in range(4):
        for g in range(_SCW // _L):
            idx = order_v[pl.ds(g * _L, _L)]
            idxc_v[c][pl.ds(g * _L, _L)] = idx * 4 + c
    gathers = [
        pltpu.async_copy(boxesf_hbm.at[idxc_v[c]], row_v[c], sem)
        for c in range(4)
    ] + [pltpu.async_copy(scores_hbm.at[order_v], row_v[4], sem)]
    for cp in gathers:
        cp.wait()
    outs = [
        pltpu.async_copy(row_v[c],
                         brf_hbm.at[pl.ds(c * _NPAD + base, _SCW)], sem2)
        for c in range(5)
    ]
    for cp in outs:
        cp.wait()


_reorder = functools.partial(
    pl.kernel,
    out_type=jax.ShapeDtypeStruct((5 * _NPAD,), jnp.float32),
    mesh=plsc.VectorSubcoreMesh(core_axis_name="c", subcore_axis_name="s"),
    scratch_types=(
        [pltpu.VMEM((_SCW,), jnp.int32)] * 5
        + [pltpu.VMEM((_SCW,), jnp.float32)] * 5
        + [pltpu.SemaphoreType.DMA, pltpu.SemaphoreType.DMA]
    ),
)(_reorder_body)


def _nms_body(br_ref, out_ref, keep_ref):
    cx = br_ref[0:1, :]
    cy = br_ref[1:2, :]
    w = br_ref[2:3, :]
    h = br_ref[3:4, :]
    s = br_ref[4:5, :]
    x1r = cx - w * 0.5
    y1r = cy - h * 0.5
    x2r = cx + w * 0.5
    y2r = cy + h * 0.5
    area_r = jnp.maximum(x2r - x1r, 0.0) * jnp.maximum(y2r - y1r, 0.0)

    col = jax.lax.broadcasted_iota(jnp.int32, (1, _NPAD), 1)
    keep_ref[...] = (col < _N).astype(jnp.float32)

    row_i = jax.lax.broadcasted_iota(jnp.int32, (_B, _B), 0)
    col_i = jax.lax.broadcasted_iota(jnp.int32, (_B, _B), 1)
    tri = (row_i < col_i).astype(jnp.float32)

    for b in range(_NB):
        start = b * _B

        rows4 = br_ref[0:4, start:start + _B]
        blk = jnp.transpose(rows4, (1, 0))
        cxc = jax.lax.slice(blk, (0, 0), (_B, 1))
        cyc = jax.lax.slice(blk, (0, 1), (_B, 2))
        wc = jax.lax.slice(blk, (0, 2), (_B, 3))
        hc = jax.lax.slice(blk, (0, 3), (_B, 4))
        x1c = cxc - wc * 0.5
        y1c = cyc - hc * 0.5
        x2c = cxc + wc * 0.5
        y2c = cyc + hc * 0.5
        area_c = jnp.maximum(x2c - x1c, 0.0) * jnp.maximum(y2c - y1c, 0.0)

        x1s = jax.lax.slice(x1r, (0, start), (1, _NPAD))
        y1s = jax.lax.slice(y1r, (0, start), (1, _NPAD))
        x2s = jax.lax.slice(x2r, (0, start), (1, _NPAD))
        y2s = jax.lax.slice(y2r, (0, start), (1, _NPAD))
        areas = jax.lax.slice(area_r, (0, start), (1, _NPAD))
        xx1 = jnp.maximum(x1c, x1s)
        yy1 = jnp.maximum(y1c, y1s)
        xx2 = jnp.minimum(x2c, x2s)
        yy2 = jnp.minimum(y2c, y2s)
        inter = jnp.maximum(xx2 - xx1, 0.0) * jnp.maximum(yy2 - yy1, 0.0)
        union = area_c + areas - inter
        ovl = (inter / (union + _EPS) > _T).astype(jnp.float32)

        m_tri = jax.lax.slice(ovl, (0, 0), (_B, _B)) * tri
        kb0 = keep_ref[0:1, start:start + _B]

        def nxt(k, m_tri=m_tri, kb0=kb0):
            cnt = jax.lax.dot_general(
                k, m_tri, (((1,), (0,)), ((), ())),
                preferred_element_type=jnp.float32)
            return kb0 * (cnt < 0.5).astype(jnp.float32)

        def cond(c):
            k, kn = c
            return jnp.sum(jnp.abs(k - kn)) > 0.0

        def body(c, nxt=nxt):
            _, k = c
            return (k, nxt(k))

        _, kb = jax.lax.while_loop(cond, body, (kb0, nxt(kb0)))
        keep_ref[0:1, start:start + _B] = kb

        if b < _NB - 1:
            cnt = jax.lax.dot_general(
                kb, ovl, (((1,), (0,)), ((), ())),
                preferred_element_type=jnp.float32)
            cnt_later = jax.lax.slice(cnt, (0, _B), (1, _NPAD - start))
            kw = keep_ref[0:1, start + _B:_NPAD]
            keep_ref[0:1, start + _B:_NPAD] = (
                kw * (1.0 - jnp.minimum(cnt_later, 1.0)))

    keep = keep_ref[...]
    dets_rows = jnp.concatenate(
        [s * keep, x1r * keep, y1r * keep, x2r * keep, y2r * keep], axis=0)
    dets = jnp.transpose(dets_rows, (1, 0))
    out_ref[...] = jax.lax.slice(dets, (0, 0), (_N, 5))


def kernel(boxes, scores):
    order = jnp.argsort(-scores).astype(jnp.int32)
    order_p = jnp.zeros((_NPAD,), jnp.int32).at[:_N].set(order)
    brf = _reorder(boxes.reshape(-1), scores, order_p)
    br = brf.reshape(5, _NPAD)
    return pl.pallas_call(
        _nms_body,
        out_shape=jax.ShapeDtypeStruct((_N, 5), jnp.float32),
        scratch_shapes=[
            pltpu.VMEM((1, _NPAD), jnp.float32),
        ],
    )(br)


# speedup vs baseline: 130.1715x; 1.0022x over previous
import functools

import jax
import jax.numpy as jnp
from jax import lax
from jax.experimental import pallas as pl
from jax.experimental.pallas import tpu as pltpu
from jax.experimental.pallas import tpu_sc as plsc

_N = 5000
_B = 256
_NPAD = 5120
_NB = _NPAD // _B
_T = 0.3
_EPS = 1e-9

_NW = 32
_SCW = _NPAD // _NW
_L = 16


def _reorder_body(boxesf_hbm, scores_hbm, order_hbm, brf_hbm,
                  order_v, i0, i1, i2, i3, r0, r1, r2, r3, r4, sem, sem2):
    idxc_v = [i0, i1, i2, i3]
    row_v = [r0, r1, r2, r3, r4]
    wid = lax.axis_index("s") * 2 + lax.axis_index("c")
    base = wid * _SCW
    pltpu.sync_copy(order_hbm.at[pl.ds(base, _SCW)], order_v)
    for c in range(4):
        for g in range(_SCW // _L):
            idx = order_v[pl.ds(g * _L, _L)]
            idxc_v[c][pl.ds(g * _L, _L)] = idx * 4 + c
    gathers = [
        pltpu.async_copy(boxesf_hbm.at[idxc_v[c]], row_v[c], sem)
        for c in range(4)
    ] + [pltpu.async_copy(scores_hbm.at[order_v], row_v[4], sem)]
    for cp in gathers:
        cp.wait()
    outs = [
        pltpu.async_copy(row_v[c],
                         brf_hbm.at[pl.ds(c * _NPAD + base, _SCW)], sem2)
        for c in range(5)
    ]
    for cp in outs:
        cp.wait()


_reorder = functools.partial(
    pl.kernel,
    out_type=jax.ShapeDtypeStruct((5 * _NPAD,), jnp.float32),
    mesh=plsc.VectorSubcoreMesh(core_axis_name="c", subcore_axis_name="s"),
    scratch_types=(
        [pltpu.VMEM((_SCW,), jnp.int32)] * 5
        + [pltpu.VMEM((_SCW,), jnp.float32)] * 5
        + [pltpu.SemaphoreType.DMA, pltpu.SemaphoreType.DMA]
    ),
)(_reorder_body)


def _nms_body(br_ref, out_ref, keep_ref):
    cx = br_ref[0:1, :]
    cy = br_ref[1:2, :]
    w = br_ref[2:3, :]
    h = br_ref[3:4, :]
    s = br_ref[4:5, :]
    x1r = cx - w * 0.5
    y1r = cy - h * 0.5
    x2r = cx + w * 0.5
    y2r = cy + h * 0.5
    area_r = jnp.maximum(x2r - x1r, 0.0) * jnp.maximum(y2r - y1r, 0.0)

    col = jax.lax.broadcasted_iota(jnp.int32, (1, _NPAD), 1)
    keep_ref[...] = (col < _N).astype(jnp.float32)

    row_i = jax.lax.broadcasted_iota(jnp.int32, (_B, _B), 0)
    col_i = jax.lax.broadcasted_iota(jnp.int32, (_B, _B), 1)
    tri = (row_i < col_i).astype(jnp.float32)

    for b in range(_NB):
        start = b * _B

        rows4 = br_ref[0:4, start:start + _B]
        blk = jnp.transpose(rows4, (1, 0))
        cxc = jax.lax.slice(blk, (0, 0), (_B, 1))
        cyc = jax.lax.slice(blk, (0, 1), (_B, 2))
        wc = jax.lax.slice(blk, (0, 2), (_B, 3))
        hc = jax.lax.slice(blk, (0, 3), (_B, 4))
        x1c = cxc - wc * 0.5
        y1c = cyc - hc * 0.5
        x2c = cxc + wc * 0.5
        y2c = cyc + hc * 0.5
        area_c = jnp.maximum(x2c - x1c, 0.0) * jnp.maximum(y2c - y1c, 0.0)

        x1s = jax.lax.slice(x1r, (0, start), (1, _NPAD))
        y1s = jax.lax.slice(y1r, (0, start), (1, _NPAD))
        x2s = jax.lax.slice(x2r, (0, start), (1, _NPAD))
        y2s = jax.lax.slice(y2r, (0, start), (1, _NPAD))
        areas = jax.lax.slice(area_r, (0, start), (1, _NPAD))
        xx1 = jnp.maximum(x1c, x1s)
        yy1 = jnp.maximum(y1c, y1s)
        xx2 = jnp.minimum(x2c, x2s)
        yy2 = jnp.minimum(y2c, y2s)
        inter = jnp.maximum(xx2 - xx1, 0.0) * jnp.maximum(yy2 - yy1, 0.0)
        union = area_c + areas - inter
        ovl = (inter / (union + _EPS) > _T).astype(jnp.float32)

        m_tri = jax.lax.slice(ovl, (0, 0), (_B, _B)) * tri
        kb0 = keep_ref[0:1, start:start + _B]

        def nxt(k, m_tri=m_tri, kb0=kb0):
            cnt = jax.lax.dot_general(
                k, m_tri, (((1,), (0,)), ((), ())),
                preferred_element_type=jnp.float32)
            return kb0 * (cnt < 0.5).astype(jnp.float32)

        def cond(c):
            k, kn = c
            return jnp.any(k != kn)

        def body(c, nxt=nxt):
            _, kn = c
            k2 = nxt(kn)
            return (k2, nxt(k2))

        _, kb = jax.lax.while_loop(cond, body, (kb0, nxt(kb0)))
        keep_ref[0:1, start:start + _B] = kb

        if b < _NB - 1:
            cnt = jax.lax.dot_general(
                kb, ovl, (((1,), (0,)), ((), ())),
                preferred_element_type=jnp.float32)
            cnt_later = jax.lax.slice(cnt, (0, _B), (1, _NPAD - start))
            kw = keep_ref[0:1, start + _B:_NPAD]
            keep_ref[0:1, start + _B:_NPAD] = (
                kw * (1.0 - jnp.minimum(cnt_later, 1.0)))

    keep = keep_ref[...]
    dets_rows = jnp.concatenate(
        [s * keep, x1r * keep, y1r * keep, x2r * keep, y2r * keep], axis=0)
    dets = jnp.transpose(dets_rows, (1, 0))
    out_ref[...] = jax.lax.slice(dets, (0, 0), (_N, 5))


def kernel(boxes, scores):
    order = jnp.argsort(-scores).astype(jnp.int32)
    order_p = jnp.zeros((_NPAD,), jnp.int32).at[:_N].set(order)
    brf = _reorder(boxes.reshape(-1), scores, order_p)
    br = brf.reshape(5, _NPAD)
    return pl.pallas_call(
        _nms_body,
        out_shape=jax.ShapeDtypeStruct((_N, 5), jnp.float32),
        scratch_shapes=[
            pltpu.VMEM((1, _NPAD), jnp.float32),
        ],
    )(br)
